# Initial kernel scaffold; baseline (speedup 1.0000x reference)
import jax
import jax.numpy as jnp
from jax.experimental import pallas as pl


def kernel(x, edge_index, W0, b0, p1, W1, b1, p2, W2, b2, Wu0, bu0, Wu1, bu1):
    raise NotImplementedError("write your pallas kernel here")


# speedup vs baseline: 1.1650x; 1.1650x over previous
import functools
import math

import jax
import jax.numpy as jnp
from jax.experimental import pallas as pl


def _rup(v: int, m: int) -> int:
    return (v + m - 1) // m * m


def _proj_body(x_ref, w_ref, o_ref):
    o_ref[...] = jnp.dot(x_ref[...], w_ref[...],
                         preferred_element_type=jnp.float32)


def _proj(x, w):
    return pl.pallas_call(
        _proj_body,
        out_shape=jax.ShapeDtypeStruct((x.shape[0], w.shape[1]), jnp.float32),
    )(x, w)


def _gcn1_fin_body(z_ref, y_ref, dinv_ref, b_ref, p_ref, x1_ref, sc_ref):
    dinv = dinv_ref[...]
    zz = z_ref[...] + 2.0 * y_ref[...]
    x1 = jnp.maximum(dinv * zz + b_ref[...], 0.0)
    x1_ref[...] = x1
    p = p_ref[...]
    pn = jnp.sqrt(jnp.sum(p * p))
    sc_ref[...] = jnp.tanh(
        jnp.dot(x1, p, preferred_element_type=jnp.float32) / pn)


def _gcn1_fin(z, y, dinv, b, p):
    n, h = z.shape
    return pl.pallas_call(
        _gcn1_fin_body,
        out_shape=(jax.ShapeDtypeStruct((n, h), jnp.float32),
                   jax.ShapeDtypeStruct((n, 1), jnp.float32)),
    )(z, y, dinv, b, p)


def _mv_body(a_ref, y_ref, dinv_ref, b_ref, o_ref, *, bm, add_id):
    i = pl.program_id(0)
    acc = jnp.dot(a_ref[...], y_ref[...], preferred_element_type=jnp.float32)
    if add_id:
        acc = acc + y_ref[pl.ds(i * bm, bm), :]
    o_ref[...] = jnp.maximum(dinv_ref[...] * acc + b_ref[...], 0.0)


def _mv_score_body(a_ref, y_ref, dinv_ref, b_ref, p_ref, o_ref, sc_ref,
                   *, bm, add_id, valid):
    i = pl.program_id(0)
    acc = jnp.dot(a_ref[...], y_ref[...], preferred_element_type=jnp.float32)
    if add_id:
        acc = acc + y_ref[pl.ds(i * bm, bm), :]
    o = jnp.maximum(dinv_ref[...] * acc + b_ref[...], 0.0)
    o_ref[...] = o
    p = p_ref[...]
    pn = jnp.sqrt(jnp.sum(p * p))
    s = jnp.tanh(jnp.dot(o, p, preferred_element_type=jnp.float32) / pn)
    rows = i * bm + jax.lax.broadcasted_iota(jnp.int32, (bm, 1), 0)
    sc_ref[...] = jnp.where(rows < valid, s, -2.0)


def _gcn_mv(a, y, dinv, b, *, bm, add_id, p=None, valid=0):
    m, k = a.shape
    h = y.shape[1]
    grid = (m // bm,)
    a_spec = pl.BlockSpec((bm, k), lambda i: (i, 0))
    full = lambda s: pl.BlockSpec(s, lambda i: (0, 0))
    d_spec = pl.BlockSpec((bm, 1), lambda i: (i, 0))
    o_spec = pl.BlockSpec((bm, h), lambda i: (i, 0))
    if p is None:
        return pl.pallas_call(
            functools.partial(_mv_body, bm=bm, add_id=add_id),
            grid=grid,
            in_specs=[a_spec, full((k, h)), d_spec, full((1, h))],
            out_specs=o_spec,
            out_shape=jax.ShapeDtypeStruct((m, h), jnp.float32),
        )(a, y, dinv, b)
    return pl.pallas_call(
        functools.partial(_mv_score_body, bm=bm, add_id=add_id, valid=valid),
        grid=grid,
        in_specs=[a_spec, full((k, h)), d_spec, full((1, h)), full((h, 1))],
        out_specs=(o_spec, pl.BlockSpec((bm, 1), lambda i: (i, 0))),
        out_shape=(jax.ShapeDtypeStruct((m, h), jnp.float32),
                   jax.ShapeDtypeStruct((m, 1), jnp.float32)),
    )(a, y, dinv, b, p)


def _sq_body(l_ref, r_ref, o_ref, deg_ref, *, bm, bn, nk, diag_val, deg_bias,
             valid):
    i, j, kk = pl.program_id(0), pl.program_id(1), pl.program_id(2)
    part = jnp.dot(l_ref[...], r_ref[...], preferred_element_type=jnp.float32)

    @pl.when(kk == 0)
    def _():
        o_ref[...] = jnp.zeros((bm, bn), jnp.float32)

    o_ref[...] += part

    @pl.when(kk == nk - 1)
    def _():
        rows = i * bm + jax.lax.broadcasted_iota(jnp.int32, (bm, bn), 0)
        cols = j * bn + jax.lax.broadcasted_iota(jnp.int32, (bm, bn), 1)
        tile = jnp.where((rows == cols) & (rows < valid), diag_val, o_ref[...])
        o_ref[...] = tile

        @pl.when(j == 0)
        def _():
            deg_ref[...] = jnp.full((bm, 1), deg_bias, jnp.float32)

        deg_ref[...] += jnp.sum(tile, axis=1, keepdims=True)


def _sq(lhs, rhs, *, bm, bn, bk, diag_val, deg_bias, valid):
    m, k = lhs.shape
    n = rhs.shape[1]
    nk = k // bk
    return pl.pallas_call(
        functools.partial(_sq_body, bm=bm, bn=bn, nk=nk, diag_val=diag_val,
                          deg_bias=deg_bias, valid=valid),
        grid=(m // bm, n // bn, nk),
        in_specs=[pl.BlockSpec((bm, bk), lambda i, j, kk: (i, kk)),
                  pl.BlockSpec((bk, bn), lambda i, j, kk: (kk, j))],
        out_specs=(pl.BlockSpec((bm, bn), lambda i, j, kk: (i, j)),
                   pl.BlockSpec((bm, 1), lambda i, j, kk: (i, 0))),
        out_shape=(jax.ShapeDtypeStruct((m, n), jnp.float32),
                   jax.ShapeDtypeStruct((m, 1), jnp.float32)),
    )(lhs, rhs)


def _fin_body(z_ref, y_ref, dinv_ref, b_ref, o_ref, *, ncls):
    zz = z_ref[...] + 2.0 * y_ref[...]
    lo = dinv_ref[...] * zz + b_ref[...]
    col = jax.lax.broadcasted_iota(jnp.int32, lo.shape, 1)
    neg = jnp.where(col < ncls, lo, -jnp.inf)
    m = jnp.max(neg, axis=1, keepdims=True)
    ex = jnp.where(col < ncls, jnp.exp(neg - m), 0.0)
    lse = m + jnp.log(jnp.sum(ex, axis=1, keepdims=True))
    o_ref[...] = lo - lse


def _fin(z, y, dinv, b, ncls):
    n, h = z.shape
    return pl.pallas_call(
        functools.partial(_fin_body, ncls=ncls),
        out_shape=jax.ShapeDtypeStruct((n, h), jnp.float32),
    )(z, y, dinv, b)


def kernel(x, edge_index, W0, b0, p1, W1, b1, p2, W2, b2, Wu0, bu0, Wu1, bu1):
    n, _ = x.shape
    hid = W0.shape[1]
    ncls = Wu1.shape[1]
    k1 = int(math.ceil(0.5 * n))
    k2 = int(math.ceil(0.5 * k1))
    bm1 = min(1024, _rup(k1, 16))
    k1p = _rup(k1, bm1)
    bm2 = min(512, _rup(k2, 16))
    k2p = _rup(k2, bm2)
    np_ = _rup(n, 256)
    ncp = _rup(ncls, 8)

    src = edge_index[0]
    dst = edge_index[1]
    f32 = jnp.float32

    indeg = jnp.zeros((n,), f32).at[dst].add(1.0)
    deg1 = indeg + 2.0
    dinv1 = jnp.where(deg1 > 0, jax.lax.rsqrt(deg1), 0.0)[:, None]

    xw = _proj(x, W0)
    y1 = dinv1 * xw
    z1 = jnp.zeros((n, hid), f32).at[dst].add(y1[src])
    x1, sc1 = _gcn1_fin(z1, y1, dinv1, b0.reshape(1, hid),
                        p1.reshape(hid, 1))
    sc1 = sc1[:, 0]

    vals1, perm1 = jax.lax.top_k(sc1, k1)
    perm1p = jnp.concatenate([perm1, jnp.zeros((k1p - k1,), jnp.int32)])
    vals1p = jnp.concatenate([vals1, jnp.zeros((k1p - k1,), f32)])
    xp2 = x1[perm1p] * vals1p[:, None]

    inv1 = jnp.full((n,), -1, jnp.int32).at[perm1].set(
        jnp.arange(k1, dtype=jnp.int32))
    ri = inv1[dst]
    rmask = ri >= 0
    ci = inv1[src]
    cmask = ci >= 0
    ar1 = jnp.arange(k1, dtype=jnp.int32)
    r1 = (jnp.zeros((k1p, np_), f32)
          .at[jnp.where(rmask, ri, 0), src].add(jnp.where(rmask, 1.0, 0.0))
          .at[ar1, perm1].add(1.0))
    c1 = (jnp.zeros((np_, k1p), f32)
          .at[dst, jnp.where(cmask, ci, 0)].add(jnp.where(cmask, 1.0, 0.0))
          .at[perm1, ar1].add(1.0))

    bk1 = min(2048, np_)
    a2pi, deg2 = _sq(r1.astype(jnp.bfloat16), c1.astype(jnp.bfloat16),
                     bm=bm1, bn=bm1, bk=bk1, diag_val=1.0, deg_bias=1.0,
                     valid=k1)
    dinv2 = jnp.where(deg2 > 0, jax.lax.rsqrt(deg2), 0.0)

    y2 = dinv2 * _proj(xp2, W1)
    x2, sc2 = _gcn_mv(a2pi, y2, dinv2, b1.reshape(1, hid), bm=bm1,
                      add_id=True, p=p2.reshape(hid, 1), valid=k1)
    sc2 = sc2[:, 0]

    vals2, perm2 = jax.lax.top_k(sc2, k2)
    perm2p = jnp.concatenate([perm2, jnp.zeros((k2p - k2,), jnp.int32)])
    vals2p = jnp.concatenate([vals2, jnp.zeros((k2p - k2,), f32)])
    mask2 = (jnp.arange(k2p) < k2)[:, None]
    xp3 = x2[perm2p] * vals2p[:, None]

    r2 = jnp.where(mask2, a2pi[perm2p, :], 0.0)
    c2 = jnp.where(mask2, a2pi[:, perm2p].T, 0.0).T

    a3h, deg3 = _sq(r2.astype(jnp.bfloat16), c2.astype(jnp.bfloat16),
                    bm=bm2, bn=bm2, bk=k1p, diag_val=2.0, deg_bias=0.0,
                    valid=k2)
    dinv3 = jnp.where(deg3 > 0, jax.lax.rsqrt(deg3), 0.0)

    y3 = dinv3 * _proj(xp3, W2)
    x3 = _gcn_mv(a3h, y3, dinv3, b2.reshape(1, hid), bm=bm2, add_id=False)

    xw_u = _proj(x2, Wu0[:hid]).at[perm2].add(_proj(x3, Wu0[hid:])[:k2])
    yu = dinv2 * xw_u
    xu = _gcn_mv(a2pi, yu, dinv2, bu0.reshape(1, hid), bm=bm1, add_id=True)

    wu1t = jnp.zeros((hid, ncp), f32).at[:, :ncls].set(Wu1[:hid])
    wu1b = jnp.zeros((hid, ncp), f32).at[:, :ncls].set(Wu1[hid:])
    bu1p = jnp.zeros((1, ncp), f32).at[0, :ncls].set(bu1)
    xwf = _proj(x1, wu1t).at[perm1].add(_proj(xu, wu1b)[:k1])
    yf = dinv1 * xwf
    zf = jnp.zeros((n, ncp), f32).at[dst].add(yf[src])
    out = _fin(zf, yf, dinv1, bu1p, ncls)
    return out[:, :ncls]
